# baseline (speedup 1.0000x reference)
import functools

import numpy as np
import jax
import jax.numpy as jnp
from jax import lax
from jax.experimental import pallas as pl
from jax.experimental.pallas import tpu as pltpu
from jax.experimental.pallas import tpu_sc as plsc

_NC = 2
_NS = 16
_NW = _NC * _NS
_L = 16
_BL = 128


def _positional_encoding(length: int, d_model: int) -> np.ndarray:
    positions = np.arange(length)[:, None]
    dims = np.arange(d_model)[None, :]
    angle_rates = 1.0 / np.power(10000.0, 2 * (dims // 2) / np.float32(d_model))
    angle_rads = positions * angle_rates
    pos = np.zeros((length, d_model), dtype=np.float32)
    pos[:, 0::2] = np.sin(angle_rads[:, 0::2])
    pos[:, 1::2] = np.cos(angle_rads[:, 1::2])
    return pos


def _make_sc_kernel(B: int, S: int, D: int, DP: int):
    scale = float(np.sqrt(np.float32(D)))
    dhi_n = D // 8
    kv_n = _BL // _L

    mesh = plsc.VectorSubcoreMesh(core_axis_name="c", subcore_axis_name="s")

    @functools.partial(
        pl.kernel,
        mesh=mesh,
        out_type=jax.ShapeDtypeStruct((S, dhi_n, _NW, 8, _BL), jnp.float32),
        compiler_params=pltpu.CompilerParams(
            use_tc_tiling_on_sc=False, needs_layout_passes=False),
        scratch_types=[
            pltpu.VMEM((S, _BL), jnp.int32),
            pltpu.VMEM((_BL, DP), jnp.float32),
            pltpu.VMEM((_BL, DP), jnp.float32),
            pltpu.VMEM((_BL, DP), jnp.float32),
            pltpu.VMEM((dhi_n, 8, _BL), jnp.float32),
            pltpu.VMEM((dhi_n, 8, _BL), jnp.float32),
            pltpu.VMEM((dhi_n, 8, _BL), jnp.float32),
            pltpu.VMEM((S, D), jnp.float32),
            pltpu.SemaphoreType.DMA,
            pltpu.SemaphoreType.DMA,
            pltpu.SemaphoreType.DMA,
            pltpu.SemaphoreType.DMA,
            pltpu.SemaphoreType.DMA,
            pltpu.SemaphoreType.DMA,
        ],
    )
    def k(xt_hbm, table_hbm, pos_hbm, out_hbm,
          idx_v, sb0, sb1, sb2, ob0, ob1, ob2, pos_v,
          gs0, gs1, gs2, os0, os1, os2):
        wid = lax.axis_index("s") * _NC + lax.axis_index("c")
        sbufs = (sb0, sb1, sb2)
        obufs = (ob0, ob1, ob2)
        gsems = (gs0, gs1, gs2)
        osems = (os0, os1, os2)

        pltpu.sync_copy(pos_hbm, pos_v)
        pltpu.sync_copy(xt_hbm.at[:, pl.ds(wid * _BL, _BL)], idx_v)

        lanes = [jnp.arange(_L, dtype=jnp.int32) + _L * kv for kv in range(kv_n)]

        def gather(s, slot):
            pltpu.async_copy(table_hbm.at[idx_v.at[s]], sbufs[slot], gsems[slot])

        def wait_gather(s, slot):
            pltpu.make_async_copy(
                table_hbm.at[idx_v.at[s]], sbufs[slot], gsems[slot]).wait()

        def out_copy(s, slot):
            return pltpu.make_async_copy(
                obufs[slot], out_hbm.at[s, :, wid], osems[slot])

        gather(0, 0)
        gather(1, 1)
        gather(2, 2)

        def step(s, slot):
            sbuf = sbufs[slot]
            obuf = obufs[slot]
            wait_gather(s, slot)

            @pl.when(s >= 3)
            def _():
                out_copy(s - 3, slot).wait()

            def dh2_body(dh2, carry):
                pvec = pos_v[s, pl.ds(dh2 * _L, _L)]
                for dl in range(_L):
                    d = dh2 * _L + dl
                    ps = pvec[dl]
                    col = jnp.full((_L,), d, dtype=jnp.int32)
                    dhi = dh2 * 2 + dl // 8
                    for kv in range(kv_n):
                        v = plsc.load_gather(sbuf, [lanes[kv], col])
                        obuf[dhi, dl % 8, pl.ds(_L * kv, _L)] = v * scale + ps
                return carry

            lax.fori_loop(0, D // _L, dh2_body, 0)
            out_copy(s, slot).start()

            @pl.when(s + 3 < S)
            def _():
                gather(s + 3, slot)

        def tri_body(j, carry):
            for k in range(3):
                s = 3 * j + k

                @pl.when(s < S)
                def _():
                    step(s, k)

            return carry

        lax.fori_loop(0, (S + 2) // 3, tri_body, 0)
        for s in range(S - 3, S):
            out_copy(s, s % 3).wait()

    return k


def kernel(x, table):
    B, S = x.shape
    V, D = table.shape
    DP = 2 * D
    pos = jnp.asarray(_positional_encoding(S, D))
    table_p = jnp.pad(table, ((0, 0), (0, DP - D)))
    xt = x.T.astype(jnp.int32)
    k = _make_sc_kernel(B, S, D, DP)
    out5 = k(xt, table_p, pos)
    return jnp.transpose(out5, (2, 4, 0, 1, 3)).reshape(B, S, D)


# speedup vs baseline: 1.5518x; 1.5518x over previous
import functools

import numpy as np
import jax
import jax.numpy as jnp
from jax import lax
from jax.experimental import pallas as pl
from jax.experimental.pallas import tpu as pltpu
from jax.experimental.pallas import tpu_sc as plsc

_NC = 2
_NS = 16
_NW = _NC * _NS
_L = 16


def _positional_encoding(length: int, d_model: int) -> np.ndarray:
    positions = np.arange(length)[:, None]
    dims = np.arange(d_model)[None, :]
    angle_rates = 1.0 / np.power(10000.0, 2 * (dims // 2) / np.float32(d_model))
    angle_rads = positions * angle_rates
    pos = np.zeros((length, d_model), dtype=np.float32)
    pos[:, 0::2] = np.sin(angle_rads[:, 0::2])
    pos[:, 1::2] = np.cos(angle_rads[:, 1::2])
    return pos


def _make_sc_kernel(B: int, S: int, D: int, DP: int):
    rows_per_w = B // _NW
    half = S // 2
    scale = float(np.sqrt(np.float32(D)))
    groups = D // _L

    mesh = plsc.VectorSubcoreMesh(core_axis_name="c", subcore_axis_name="s")

    @functools.partial(
        pl.kernel,
        mesh=mesh,
        out_type=jax.ShapeDtypeStruct((B * S, DP), jnp.float32),
        compiler_params=pltpu.CompilerParams(use_tc_tiling_on_sc=False),
        scratch_types=[
            pltpu.VMEM((2 * rows_per_w, half), jnp.int32),
            pltpu.VMEM((S, DP), jnp.float32),
            pltpu.VMEM((S, DP), jnp.float32),
            pltpu.VMEM((S, D), jnp.float32),
            pltpu.VMEM((S, D), jnp.float32),
            pltpu.VMEM((S, D), jnp.float32),
            pltpu.SemaphoreType.DMA,
            pltpu.SemaphoreType.DMA,
            pltpu.SemaphoreType.DMA,
            pltpu.SemaphoreType.DMA,
        ],
    )
    def k(x_hbm, table_hbm, pos_hbm, out_hbm,
          idx_v, buf0, buf1, ob0, ob1, pos_v, gs0, gs1, os0, os1):
        wid = lax.axis_index("s") * _NC + lax.axis_index("c")
        bufs = (buf0, buf1)
        obufs = (ob0, ob1)
        gsems = (gs0, gs1)
        osems = (os0, os1)

        pltpu.sync_copy(pos_hbm, pos_v)
        pltpu.sync_copy(x_hbm.at[wid], idx_v)
        base_b = wid * rows_per_w

        def gather_chunk(i, slot):
            pltpu.async_copy(
                table_hbm.at[idx_v.at[2 * i]],
                bufs[slot].at[pl.ds(0, half)], gsems[slot])
            pltpu.async_copy(
                table_hbm.at[idx_v.at[2 * i + 1]],
                bufs[slot].at[pl.ds(half, half)], gsems[slot])

        def wait_gather(i, slot):
            pltpu.make_async_copy(
                table_hbm.at[idx_v.at[2 * i]],
                bufs[slot].at[pl.ds(0, half)], gsems[slot]).wait()
            pltpu.make_async_copy(
                table_hbm.at[idx_v.at[2 * i + 1]],
                bufs[slot].at[pl.ds(half, half)], gsems[slot]).wait()

        def out_dst(i):
            return out_hbm.at[pl.ds((base_b + i) * S, S), pl.ds(0, D)]

        def wait_out(i, slot):
            pltpu.make_async_copy(obufs[slot], out_dst(i), osems[slot]).wait()

        gather_chunk(0, 0)
        gather_chunk(1, 1)

        def step(i, slot):
            buf = bufs[slot]
            obuf = obufs[slot]
            wait_gather(i, slot)

            @pl.when(i >= 2)
            def _():
                wait_out(i - 2, slot)

            def row_body(r, carry):
                for g in range(groups):
                    sl = pl.ds(g * _L, _L)
                    obuf[r, sl] = buf[r, sl] * scale + pos_v[r, sl]
                return carry

            lax.fori_loop(0, S, row_body, 0, unroll=8)
            pltpu.async_copy(obuf, out_dst(i), osems[slot])

            @pl.when(i + 2 < rows_per_w)
            def _():
                gather_chunk(i + 2, slot)

        def pair_body(j, carry):
            step(2 * j, 0)
            step(2 * j + 1, 1)
            return carry

        lax.fori_loop(0, rows_per_w // 2, pair_body, 0)
        wait_out(rows_per_w - 2, 0)
        wait_out(rows_per_w - 1, 1)

    return k


def kernel(x, table):
    B, S = x.shape
    V, D = table.shape
    DP = 2 * D
    pos = jnp.asarray(_positional_encoding(S, D))
    table_p = jnp.pad(table, ((0, 0), (0, DP - D)))
    x3 = x.reshape(_NW, (B // _NW) * 2, S // 2).astype(jnp.int32)
    k = _make_sc_kernel(B, S, D, DP)
    out = k(x3, table_p, pos)
    return out[:, :D].reshape(B, S, D)
